# Initial kernel scaffold; baseline (speedup 1.0000x reference)
import jax
import jax.numpy as jnp
from jax.experimental import pallas as pl


def kernel(x, edge_index, batch, W1a, b1a, W2a, b2a, W1b, b1b, W2b, b2b, Wl, bl, Wl2, bl2, Wo, bo):
    raise NotImplementedError("write your pallas kernel here")


# speedup vs baseline: 1.4622x; 1.4622x over previous
import functools

import jax
import jax.numpy as jnp
from jax import lax
from jax.experimental import pallas as pl
from jax.experimental.pallas import tpu as pltpu
from jax.experimental.pallas import tpu_sc as plsc

N = 10000
E = 320000
D = 128
G = 64
OUT = 16

NC = 2
NS = 16
L = 16
NW = NC * NS
EPW = E // NW
C_G = 80
RPT = 320
PADROW = RPT
ACCR = RPT + 8
KB = 128
CD = 2000

_MESH = dict(core_axis_name="c", subcore_axis_name="s")
_SC_PARAMS = pltpu.CompilerParams(needs_layout_passes=False)


def _proj_body(x_ref, w1_ref, b1_ref, a_ref, b_ref):
    xv = x_ref[...]
    wt = w1_ref[0:D, :]
    wb = w1_ref[D:2 * D, :]
    a_ref[...] = (jnp.dot(xv, wt - wb, preferred_element_type=jnp.float32)
                  + b1_ref[...])
    b_ref[...] = jnp.dot(xv, wb, preferred_element_type=jnp.float32)


def _node_proj(x, W1, b1):
    return pl.pallas_call(
        _proj_body,
        out_shape=(jax.ShapeDtypeStruct((N, D), jnp.float32),
                   jax.ShapeDtypeStruct((N, D), jnp.float32)),
    )(x, W1, b1.reshape(1, D))


def _mm_body(h_ref, w_ref, b_ref, o_ref):
    o_ref[...] = (jnp.dot(h_ref[...], w_ref[...],
                          preferred_element_type=jnp.float32) + b_ref[...])


def _edge_mm(h, W2, b2):
    BM = 2560
    return pl.pallas_call(
        _mm_body,
        grid=(E // BM,),
        in_specs=[pl.BlockSpec((BM, D), lambda i: (i, 0)),
                  pl.BlockSpec((D, D), lambda i: (0, 0)),
                  pl.BlockSpec((1, D), lambda i: (0, 0))],
        out_specs=pl.BlockSpec((BM, D), lambda i: (i, 0)),
        out_shape=jax.ShapeDtypeStruct((E, D), jnp.float32),
    )(h, W2, b2.reshape(1, D))


def _head_body(h_ref, batch_ref, wl_ref, bl_ref, wl2_ref, bl2_ref,
               wo_ref, bo_ref, o_ref):
    b2d = batch_ref[...]
    gid = lax.broadcasted_iota(jnp.int32, (G, N), 0)
    oh = (b2d == gid).astype(jnp.float32)
    s = jnp.dot(oh, h_ref[...], preferred_element_type=jnp.float32)
    cnt = jnp.sum(oh, axis=1, keepdims=True)
    pooled = s / jnp.maximum(cnt, 1.0)
    z = jnp.maximum(jnp.dot(pooled, wl_ref[...],
                            preferred_element_type=jnp.float32)
                    + bl_ref[...], 0.0)
    z = jnp.maximum(jnp.dot(z, wl2_ref[...],
                            preferred_element_type=jnp.float32)
                    + bl2_ref[...], 0.0)
    z = jnp.maximum(jnp.dot(z, wo_ref[...],
                            preferred_element_type=jnp.float32)
                    + bo_ref[...], 0.0)
    zmax = jnp.max(z, axis=0, keepdims=True)
    ez = jnp.exp(z - zmax)
    o_ref[...] = ez / jnp.sum(ez, axis=0, keepdims=True)


def _head(h, batch, Wl, bl, Wl2, bl2, Wo, bo):
    return pl.pallas_call(
        _head_body,
        out_shape=jax.ShapeDtypeStruct((G, OUT), jnp.float32),
    )(h, batch.reshape(1, N), Wl, bl.reshape(1, -1),
      Wl2, bl2.reshape(1, -1), Wo, bo.reshape(1, -1))


def _gather_body(a_hbm, b_hbm, dst_hbm, src_hbm, h_hbm,
                 dstv, srcv, arows, brows, s1, s2):
    wid = lax.axis_index("s") * NC + lax.axis_index("c")
    base = pl.multiple_of(wid * EPW, 8)

    def chunk(k, carry):
        off = pl.multiple_of(base + k * C_G, 8)
        pltpu.sync_copy(dst_hbm.at[pl.ds(off, C_G)], dstv)
        pltpu.sync_copy(src_hbm.at[pl.ds(off, C_G)], srcv)
        cp1 = pltpu.async_copy(a_hbm.at[dstv], arows, s1)
        cp2 = pltpu.async_copy(b_hbm.at[srcv], brows, s2)
        cp1.wait()
        cp2.wait()

        def edge(i, c2):
            for j in range(D // L):
                sl = pl.ds(j * L, L)
                arows[i, sl] = jnp.maximum(arows[i, sl] + brows[i, sl], 0.0)
            return c2

        lax.fori_loop(0, C_G, edge, 0)
        pltpu.sync_copy(arows, h_hbm.at[pl.ds(off, C_G)])
        return carry

    lax.fori_loop(0, EPW // C_G, chunk, 0)


def _edge_gather(A, B, dst, src):
    mesh = plsc.VectorSubcoreMesh(**_MESH)
    f = pl.kernel(
        _gather_body,
        out_type=jax.ShapeDtypeStruct((E, D), jnp.float32),
        mesh=mesh,
        compiler_params=_SC_PARAMS,
        scratch_types=[
            pltpu.VMEM((C_G,), jnp.int32),
            pltpu.VMEM((C_G,), jnp.int32),
            pltpu.VMEM((C_G, D), jnp.float32),
            pltpu.VMEM((C_G, D), jnp.float32),
            pltpu.SemaphoreType.DMA,
            pltpu.SemaphoreType.DMA,
        ],
    )
    return f(A, B, dst, src)


def _scatter_body(m_hbm, dst_hbm, out_hbm, acc, dbuf, selg, selr, rows, sd):
    wid = lax.axis_index("s") * NC + lax.axis_index("c")
    lo = wid * RPT
    iota = lax.iota(jnp.int32, L)
    zf = jnp.zeros((L,), jnp.float32)
    zg = jnp.zeros((L,), jnp.int32)
    padr = jnp.full((L,), PADROW, jnp.int32)

    def zr(r, c):
        for j in range(D // L):
            acc[r, pl.ds(j * L, L)] = zf
        return c

    lax.fori_loop(0, ACCR, zr, 0)

    def initsel(v, c):
        selg[pl.ds(v * L, L)] = zg
        selr[pl.ds(v * L, L)] = padr
        return c

    lax.fori_loop(0, (KB + L) // L, initsel, 0)

    def fire(cursor):
        cp = pltpu.async_copy(m_hbm.at[selg.at[pl.ds(0, KB)]], rows, sd)
        cp.wait()

        def proc(i, c):
            r = selr[pl.ds(i, L)][0]
            for j in range(D // L):
                sl = pl.ds(j * L, L)
                acc[r, sl] = jnp.maximum(acc[r, sl], rows[i, sl])
            return c

        lax.fori_loop(0, KB, proc, 0)
        selg[pl.ds(0, L)] = selg[pl.ds(KB, L)]
        selr[pl.ds(0, L)] = selr[pl.ds(KB, L)]
        return cursor - KB

    def chunk(kc, cursor):
        off = pl.multiple_of(kc * CD, 8)
        pltpu.sync_copy(dst_hbm.at[pl.ds(off, CD)], dbuf)

        def vec(v, cur):
            d = dbuf[pl.ds(v * L, L)]
            eid = off + v * L + iota
            m = (d >= lo) & (d < lo + RPT)
            plsc.store_compressed(selg.at[pl.ds(cur, L)], eid, mask=m)
            plsc.store_compressed(selr.at[pl.ds(cur, L)], d - lo, mask=m)
            cur = cur + jnp.sum(m.astype(jnp.int32))
            return lax.cond(cur >= KB, fire, lambda c: c, cur)

        return lax.fori_loop(0, CD // L, vec, cursor)

    cursor = lax.fori_loop(0, E // CD, chunk, 0)

    def padtail(v, c):
        base = v * L
        lanes = base + iota
        m = lanes >= cursor
        selg[pl.ds(base, L)] = jnp.where(m, zg, selg[pl.ds(base, L)])
        selr[pl.ds(base, L)] = jnp.where(m, padr, selr[pl.ds(base, L)])
        return c

    lax.fori_loop(0, KB // L, padtail, 0)
    fire(0)

    rem = N - (NW - 1) * RPT

    @pl.when(wid < NW - 1)
    def _():
        pltpu.sync_copy(acc.at[pl.ds(0, RPT)], out_hbm.at[pl.ds(lo, RPT)])

    @pl.when(wid == NW - 1)
    def _():
        pltpu.sync_copy(acc.at[pl.ds(0, rem)], out_hbm.at[pl.ds(lo, rem)])


def _seg_max(m, dst):
    mesh = plsc.VectorSubcoreMesh(**_MESH)
    f = pl.kernel(
        _scatter_body,
        out_type=jax.ShapeDtypeStruct((N, D), jnp.float32),
        mesh=mesh,
        compiler_params=_SC_PARAMS,
        scratch_types=[
            pltpu.VMEM((ACCR, D), jnp.float32),
            pltpu.VMEM((CD,), jnp.int32),
            pltpu.VMEM((KB + L,), jnp.int32),
            pltpu.VMEM((KB + L,), jnp.int32),
            pltpu.VMEM((KB, D), jnp.float32),
            pltpu.SemaphoreType.DMA,
        ],
    )
    return f(m, dst)


def _conv(x, dst, src, W1, b1, W2, b2):
    A, B = _node_proj(x, W1, b1)
    h = _edge_gather(A, B, dst, src)
    m = _edge_mm(h, W2, b2)
    return _seg_max(m, dst)


def kernel(x, edge_index, batch, W1a, b1a, W2a, b2a, W1b, b1b, W2b, b2b,
           Wl, bl, Wl2, bl2, Wo, bo):
    src = edge_index[0]
    dst = edge_index[1]
    h1 = _conv(x, dst, src, W1a, b1a, W2a, b2a)
    h2 = _conv(h1, dst, src, W1b, b1b, W2b, b2b)
    return _head(h2, batch, Wl, bl, Wl2, bl2, Wo, bo)


# speedup vs baseline: 1.7989x; 1.2302x over previous
import functools

import jax
import jax.numpy as jnp
from jax import lax
from jax.experimental import pallas as pl
from jax.experimental.pallas import tpu as pltpu
from jax.experimental.pallas import tpu_sc as plsc

N = 10000
E = 320000
D = 128
G = 64
OUT = 16

NC = 2
NS = 16
L = 16
NW = NC * NS
EPW = E // NW
C_G = 80
RPT = 640
PADROW = RPT
ACCR = RPT + 8
KB = 128
CD = 2000

_MESH = dict(core_axis_name="c", subcore_axis_name="s")
_SC_PARAMS = pltpu.CompilerParams(needs_layout_passes=False)


def _proj_body(x_ref, w1_ref, b1_ref, a_ref, b_ref):
    if x_ref.shape == (NC, N, D):
        xv = jnp.maximum(x_ref[0], x_ref[1])
    else:
        xv = x_ref[...]
    wt = w1_ref[0:D, :]
    wb = w1_ref[D:2 * D, :]
    a_ref[...] = (jnp.dot(xv, wt - wb, preferred_element_type=jnp.float32)
                  + b1_ref[...])
    b_ref[...] = jnp.dot(xv, wb, preferred_element_type=jnp.float32)


def _node_proj(x, W1, b1):
    return pl.pallas_call(
        _proj_body,
        out_shape=(jax.ShapeDtypeStruct((N, D), jnp.float32),
                   jax.ShapeDtypeStruct((N, D), jnp.float32)),
    )(x, W1, b1.reshape(1, D))


def _mm_body(h_ref, w_ref, b_ref, o_ref):
    o_ref[...] = (jnp.dot(h_ref[...], w_ref[...],
                          preferred_element_type=jnp.float32) + b_ref[...])


def _edge_mm(h, W2, b2):
    BM = 2560
    return pl.pallas_call(
        _mm_body,
        grid=(E // BM,),
        in_specs=[pl.BlockSpec((BM, D), lambda i: (i, 0)),
                  pl.BlockSpec((D, D), lambda i: (0, 0)),
                  pl.BlockSpec((1, D), lambda i: (0, 0))],
        out_specs=pl.BlockSpec((BM, D), lambda i: (i, 0)),
        out_shape=jax.ShapeDtypeStruct((E, D), jnp.float32),
    )(h, W2, b2.reshape(1, D))


def _head_body(h_ref, batch_ref, wl_ref, bl_ref, wl2_ref, bl2_ref,
               wo_ref, bo_ref, o_ref):
    hv = jnp.maximum(h_ref[0], h_ref[1])
    b2d = batch_ref[...]
    gid = lax.broadcasted_iota(jnp.int32, (G, N), 0)
    oh = (b2d == gid).astype(jnp.float32)
    s = jnp.dot(oh, hv, preferred_element_type=jnp.float32)
    cnt = jnp.sum(oh, axis=1, keepdims=True)
    pooled = s / jnp.maximum(cnt, 1.0)
    z = jnp.maximum(jnp.dot(pooled, wl_ref[...],
                            preferred_element_type=jnp.float32)
                    + bl_ref[...], 0.0)
    z = jnp.maximum(jnp.dot(z, wl2_ref[...],
                            preferred_element_type=jnp.float32)
                    + bl2_ref[...], 0.0)
    z = jnp.maximum(jnp.dot(z, wo_ref[...],
                            preferred_element_type=jnp.float32)
                    + bo_ref[...], 0.0)
    zmax = jnp.max(z, axis=0, keepdims=True)
    ez = jnp.exp(z - zmax)
    o_ref[...] = ez / jnp.sum(ez, axis=0, keepdims=True)


def _head(h, batch, Wl, bl, Wl2, bl2, Wo, bo):
    return pl.pallas_call(
        _head_body,
        out_shape=jax.ShapeDtypeStruct((G, OUT), jnp.float32),
    )(h, batch.reshape(1, N), Wl, bl.reshape(1, -1),
      Wl2, bl2.reshape(1, -1), Wo, bo.reshape(1, -1))


def _gather_body(a_hbm, b_hbm, dst_hbm, src_hbm, h_hbm,
                 dstv, srcv, arows, brows, s1, s2):
    wid = lax.axis_index("s") * NC + lax.axis_index("c")
    base = pl.multiple_of(wid * EPW, 8)

    def chunk(k, carry):
        off = pl.multiple_of(base + k * C_G, 8)
        pltpu.sync_copy(dst_hbm.at[pl.ds(off, C_G)], dstv)
        pltpu.sync_copy(src_hbm.at[pl.ds(off, C_G)], srcv)
        cp1 = pltpu.async_copy(a_hbm.at[dstv], arows, s1)
        cp2 = pltpu.async_copy(b_hbm.at[srcv], brows, s2)
        cp1.wait()
        cp2.wait()

        def edge(i, c2):
            for j in range(D // L):
                sl = pl.ds(j * L, L)
                arows[i, sl] = jnp.maximum(arows[i, sl] + brows[i, sl], 0.0)
            return c2

        lax.fori_loop(0, C_G, edge, 0)
        pltpu.sync_copy(arows, h_hbm.at[pl.ds(off, C_G)])
        return carry

    lax.fori_loop(0, EPW // C_G, chunk, 0)


def _edge_gather(A, B, dst, src):
    mesh = plsc.VectorSubcoreMesh(**_MESH)
    f = pl.kernel(
        _gather_body,
        out_type=jax.ShapeDtypeStruct((E, D), jnp.float32),
        mesh=mesh,
        compiler_params=_SC_PARAMS,
        scratch_types=[
            pltpu.VMEM((C_G,), jnp.int32),
            pltpu.VMEM((C_G,), jnp.int32),
            pltpu.VMEM((C_G, D), jnp.float32),
            pltpu.VMEM((C_G, D), jnp.float32),
            pltpu.SemaphoreType.DMA,
            pltpu.SemaphoreType.DMA,
        ],
    )
    return f(A, B, dst, src)


def _scatter_body(m_hbm, dst_hbm, out_hbm, acc, db0, db1, selp, idxb, rows,
                  sd, sb0, sb1):
    c = lax.axis_index("c")
    s = lax.axis_index("s")
    lo = s * RPT
    ebase = c * (E // NC)
    iota = lax.iota(jnp.int32, L)
    zf = jnp.zeros((L,), jnp.float32)
    padp = jnp.full((L,), PADROW, jnp.int32)

    def zr(r, cc):
        for j in range(D // L):
            acc[r, pl.ds(j * L, L)] = zf
        return cc

    lax.fori_loop(0, ACCR, zr, 0)

    def initsel(v, cc):
        selp[pl.ds(v * L, L)] = padp
        return cc

    lax.fori_loop(0, (KB + L) // L, initsel, 0)

    def fire(cursor):
        def up(j, cc):
            sl = pl.ds(j * L, L)
            idxb[sl] = lax.shift_right_logical(selp[sl], 10)
            return cc

        lax.fori_loop(0, KB // L, up, 0)
        pltpu.async_copy(m_hbm.at[idxb], rows, sd).wait()

        def proc(i, cc):
            r = selp[pl.ds(i, L)][0] & 1023
            for j in range(D // L):
                sl = pl.ds(j * L, L)
                acc[r, sl] = jnp.maximum(acc[r, sl], rows[i, sl])
            return cc

        lax.fori_loop(0, KB, proc, 0)
        selp[pl.ds(0, L)] = selp[pl.ds(KB, L)]
        return cursor - KB

    def scan_chunk(db, off, cursor):
        def vec(v, cur):
            d = db[pl.ds(v * L, L)]
            msk = (d >= lo) & (d < lo + RPT)
            cnt = plsc.all_reduce_population_count(msk)[0]

            def sel(cur2):
                packed = lax.shift_left(off + v * L + iota, 10) | (d - lo)
                plsc.store_compressed(selp.at[pl.ds(cur2, L)], packed,
                                      mask=msk)
                return lax.cond(cur2 + cnt >= KB, fire,
                                lambda x: x, cur2 + cnt)

            return lax.cond(cnt > 0, sel, lambda x: x, cur)

        return lax.fori_loop(0, CD // L, vec, cursor)

    NCH = (E // NC) // CD
    cp0 = pltpu.async_copy(dst_hbm.at[pl.ds(ebase, CD)], db0, sb0)

    def pair(p, cursor):
        off0 = pl.multiple_of(ebase + (2 * p) * CD, 8)
        cp0 = pltpu.make_async_copy(dst_hbm.at[pl.ds(off0, CD)], db0, sb0)
        cp0.wait()
        off1 = pl.multiple_of(off0 + CD, 8)
        pltpu.async_copy(dst_hbm.at[pl.ds(off1, CD)], db1, sb1)
        cursor = scan_chunk(db0, off0, cursor)
        pltpu.make_async_copy(dst_hbm.at[pl.ds(off1, CD)], db1, sb1).wait()

        @pl.when(p < NCH // 2 - 1)
        def _():
            off2 = pl.multiple_of(off1 + CD, 8)
            pltpu.async_copy(dst_hbm.at[pl.ds(off2, CD)], db0, sb0)

        return scan_chunk(db1, off1, cursor)

    cursor = lax.fori_loop(0, NCH // 2, pair, 0)

    def padtail(v, cc):
        base = v * L
        msk = (base + iota) >= cursor
        selp[pl.ds(base, L)] = jnp.where(msk, padp, selp[pl.ds(base, L)])
        return cc

    lax.fori_loop(0, KB // L, padtail, 0)
    fire(0)

    rem = N - (NS - 1) * RPT

    @pl.when(s < NS - 1)
    def _():
        pltpu.sync_copy(acc.at[pl.ds(0, RPT)], out_hbm.at[c, pl.ds(lo, RPT)])

    @pl.when(s == NS - 1)
    def _():
        pltpu.sync_copy(acc.at[pl.ds(0, rem)], out_hbm.at[c, pl.ds(lo, rem)])


def _seg_max(m, dst):
    mesh = plsc.VectorSubcoreMesh(**_MESH)
    f = pl.kernel(
        _scatter_body,
        out_type=jax.ShapeDtypeStruct((NC, N, D), jnp.float32),
        mesh=mesh,
        compiler_params=_SC_PARAMS,
        scratch_types=[
            pltpu.VMEM((ACCR, D), jnp.float32),
            pltpu.VMEM((CD,), jnp.int32),
            pltpu.VMEM((CD,), jnp.int32),
            pltpu.VMEM((KB + L,), jnp.int32),
            pltpu.VMEM((KB,), jnp.int32),
            pltpu.VMEM((KB, D), jnp.float32),
            pltpu.SemaphoreType.DMA,
            pltpu.SemaphoreType.DMA,
            pltpu.SemaphoreType.DMA,
        ],
    )
    return f(m, dst)


def _conv(x, dst, src, W1, b1, W2, b2):
    A, B = _node_proj(x, W1, b1)
    h = _edge_gather(A, B, dst, src)
    m = _edge_mm(h, W2, b2)
    return _seg_max(m, dst)


def kernel(x, edge_index, batch, W1a, b1a, W2a, b2a, W1b, b1b, W2b, b2b,
           Wl, bl, Wl2, bl2, Wo, bo):
    src = edge_index[0]
    dst = edge_index[1]
    h1 = _conv(x, dst, src, W1a, b1a, W2a, b2a)
    h2 = _conv(h1, dst, src, W1b, b1b, W2b, b2b)
    return _head(h2, batch, Wl, bl, Wl2, bl2, Wo, bo)
